# baseline (speedup 1.0000x reference)
import jax
import jax.numpy as jnp
from jax import lax
from jax.experimental import pallas as pl
from jax.experimental.pallas import tpu as pltpu
from jax.experimental.pallas import tpu_sc as plsc

B = 4
S = 8192
D = 768
LANES = 16

NC = 2
NS = 16
NW = NC * NS

SPW = S // NW
C = 32
NSC = SPW // C
NUNIT = NSC * B
UPT = 8
NT = NUNIT // UPT
GROUPS = D // LANES


def _body(ids_hbm, word_hbm, pos_hbm, out_hbm,
          idx_v, rows0, rows1, pos0, pos1,
          gsem0, gsem1, psem0, psem1, wsem0, wsem1):
    wid = lax.axis_index("s") * NC + lax.axis_index("c")
    soff = wid * SPW

    rows = (rows0, rows1)
    pos = (pos0, pos1)
    gsem = (gsem0, gsem1)
    psem = (psem0, psem1)
    wsem = (wsem0, wsem1)

    for b in range(B):
        pltpu.async_copy(ids_hbm.at[b, wid], idx_v.at[b], gsem0)
    for b in range(B):
        pltpu.make_async_copy(ids_hbm.at[0, 0], idx_v.at[b], gsem0).wait()

    def issue_pos(sc, q):
        pltpu.async_copy(pos_hbm.at[pl.ds(soff + sc * C, C)], pos[q], psem[q])

    def wait_pos(q):
        pltpu.make_async_copy(pos_hbm.at[pl.ds(0, C)], pos[q], psem[q]).wait()

    def issue_gather(sc, b, p):
        pltpu.async_copy(word_hbm.at[idx_v.at[b, sc]], rows[p], gsem[p])

    def wait_gather(p):
        pltpu.make_async_copy(
            word_hbm.at[idx_v.at[0, 0]], rows[p], gsem[p]).wait()

    H = C // 2

    def issue_write_half(sc, b, p, h):
        pltpu.async_copy(
            rows[p].at[pl.ds(h * H, H)],
            out_hbm.at[pl.ds(b * S + soff + sc * C + h * H, H)], wsem[p])

    def wait_write(p):
        for _ in range(2):
            pltpu.make_async_copy(
                rows[p].at[pl.ds(0, H)], out_hbm.at[pl.ds(0, H)],
                wsem[p]).wait()

    def add_pos_half(p, q, h):
        rbuf = rows[p]
        pbuf = pos[q]

        @plsc.parallel_loop(h * H, (h + 1) * H, step=1, unroll=2)
        def row_body(r):
            for j in range(GROUPS):
                sl = pl.ds(j * LANES, LANES)
                plsc.addupdate(rbuf.at[r, sl], pbuf[r, sl])

    issue_pos(0, 0)
    issue_gather(0, 0, 0)

    def iter_body(t, carry):
        for k in range(UPT):
            p = k % 2
            q = k // 4
            sc = 2 * t + q
            b = k % 4
            if k == 0:
                issue_pos(sc + 1, 1)
                wait_pos(0)
            if k == 4:
                @pl.when(t < NT - 1)
                def _():
                    issue_pos(sc + 1, 0)
                wait_pos(1)
            if k == 0:
                @pl.when(t > 0)
                def _():
                    wait_write(1 - p)
                issue_gather(sc, b + 1, 1 - p)
            elif k == UPT - 1:
                wait_write(1 - p)
                @pl.when(t < NT - 1)
                def _():
                    issue_gather(sc + 1, 0, 1 - p)
            else:
                wait_write(1 - p)
                issue_gather(sc + (1 if k == 3 else 0), (b + 1) % 4, 1 - p)
            wait_gather(p)
            add_pos_half(p, q, 0)
            issue_write_half(sc, b, p, 0)
            add_pos_half(p, q, 1)
            issue_write_half(sc, b, p, 1)
        return carry

    lax.fori_loop(0, NT, iter_body, 0, unroll=False)

    wait_write(1)


@jax.jit
def kernel(input_ids, word_embeddings_weight, position_embeddings_weight):
    ids = jnp.reshape(input_ids.astype(jnp.int32), (B, NW, NSC, C))
    fn = pl.kernel(
        _body,
        out_type=jax.ShapeDtypeStruct((B * S, D), jnp.float32),
        mesh=plsc.VectorSubcoreMesh(core_axis_name="c", subcore_axis_name="s"),
        scratch_types=[
            pltpu.VMEM((B, NSC, C), jnp.int32),
            pltpu.VMEM((C, D), jnp.float32),
            pltpu.VMEM((C, D), jnp.float32),
            pltpu.VMEM((C, D), jnp.float32),
            pltpu.VMEM((C, D), jnp.float32),
            pltpu.SemaphoreType.DMA,
            pltpu.SemaphoreType.DMA,
            pltpu.SemaphoreType.DMA,
            pltpu.SemaphoreType.DMA,
            pltpu.SemaphoreType.DMA,
            pltpu.SemaphoreType.DMA,
        ],
    )
    out = fn(ids, word_embeddings_weight, position_embeddings_weight)
    return jnp.reshape(out, (B, S, D))


# speedup vs baseline: 1.0605x; 1.0605x over previous
import jax
import jax.numpy as jnp
from jax import lax
from jax.experimental import pallas as pl
from jax.experimental.pallas import tpu as pltpu
from jax.experimental.pallas import tpu_sc as plsc

B = 4
S = 8192
D = 768
LANES = 16

NC = 2
NS = 16
NW = NC * NS

SPW = S // NW
C = 32
NSC = SPW // C
NUNIT = NSC * B
UPT = 8
NT = NUNIT // UPT
GROUPS = D // LANES


def _body(ids_hbm, word_hbm, pos_hbm, out_hbm,
          idx_v, rows0, rows1, pos0, pos1,
          gsem0, gsem1, psem0, psem1, wsem0, wsem1):
    wid = lax.axis_index("s") * NC + lax.axis_index("c")
    soff = wid * SPW

    rows = (rows0, rows1)
    pos = (pos0, pos1)
    gsem = (gsem0, gsem1)
    psem = (psem0, psem1)
    wsem = (wsem0, wsem1)

    for b in range(B):
        pltpu.async_copy(ids_hbm.at[b, wid], idx_v.at[b], gsem0)
    for b in range(B):
        pltpu.make_async_copy(ids_hbm.at[0, 0], idx_v.at[b], gsem0).wait()

    def issue_pos(sc, q):
        pltpu.async_copy(pos_hbm.at[pl.ds(soff + sc * C, C)], pos[q], psem[q])

    def wait_pos(q):
        pltpu.make_async_copy(pos_hbm.at[pl.ds(0, C)], pos[q], psem[q]).wait()

    def issue_gather(sc, b, p):
        pltpu.async_copy(word_hbm.at[idx_v.at[b, sc]], rows[p], gsem[p])

    def wait_gather(p):
        pltpu.make_async_copy(
            word_hbm.at[idx_v.at[0, 0]], rows[p], gsem[p]).wait()

    H = C // 2

    def issue_write_half(sc, b, p, h):
        pltpu.async_copy(
            rows[p].at[pl.ds(h * H, H)],
            out_hbm.at[pl.ds(b * S + soff + sc * C + h * H, H)], wsem[p])

    def wait_write(p):
        for _ in range(2):
            pltpu.make_async_copy(
                rows[p].at[pl.ds(0, H)], out_hbm.at[pl.ds(0, H)],
                wsem[p]).wait()

    def add_pos_half(p, q, h):
        rbuf = rows[p]
        pbuf = pos[q]

        def row_body(r, carry):
            for j in range(GROUPS):
                sl = pl.ds(j * LANES, LANES)
                rbuf[r, sl] = rbuf[r, sl] + pbuf[r, sl]
            return carry

        lax.fori_loop(h * H, (h + 1) * H, row_body, 0, unroll=False)

    issue_pos(0, 0)
    issue_gather(0, 0, 0)

    def iter_body(t, carry):
        for k in range(UPT):
            p = k % 2
            q = k // 4
            sc = 2 * t + q
            b = k % 4
            if k == 0:
                issue_pos(sc + 1, 1)
                wait_pos(0)
            if k == 4:
                @pl.when(t < NT - 1)
                def _():
                    issue_pos(sc + 1, 0)
                wait_pos(1)
            if k == 0:
                @pl.when(t > 0)
                def _():
                    wait_write(1 - p)
                issue_gather(sc, b + 1, 1 - p)
            elif k == UPT - 1:
                wait_write(1 - p)
                @pl.when(t < NT - 1)
                def _():
                    issue_gather(sc + 1, 0, 1 - p)
            else:
                wait_write(1 - p)
                issue_gather(sc + (1 if k == 3 else 0), (b + 1) % 4, 1 - p)
            wait_gather(p)
            add_pos_half(p, q, 0)
            issue_write_half(sc, b, p, 0)
            add_pos_half(p, q, 1)
            issue_write_half(sc, b, p, 1)
        return carry

    lax.fori_loop(0, NT, iter_body, 0, unroll=False)

    wait_write(1)


@jax.jit
def kernel(input_ids, word_embeddings_weight, position_embeddings_weight):
    ids = jnp.reshape(input_ids.astype(jnp.int32), (B, NW, NSC, C))
    fn = pl.kernel(
        _body,
        out_type=jax.ShapeDtypeStruct((B * S, D), jnp.float32),
        mesh=plsc.VectorSubcoreMesh(core_axis_name="c", subcore_axis_name="s"),
        scratch_types=[
            pltpu.VMEM((B, NSC, C), jnp.int32),
            pltpu.VMEM((C, D), jnp.float32),
            pltpu.VMEM((C, D), jnp.float32),
            pltpu.VMEM((C, D), jnp.float32),
            pltpu.VMEM((C, D), jnp.float32),
            pltpu.SemaphoreType.DMA,
            pltpu.SemaphoreType.DMA,
            pltpu.SemaphoreType.DMA,
            pltpu.SemaphoreType.DMA,
            pltpu.SemaphoreType.DMA,
            pltpu.SemaphoreType.DMA,
        ],
    )
    out = fn(ids, word_embeddings_weight, position_embeddings_weight)
    return jnp.reshape(out, (B, S, D))


# speedup vs baseline: 1.1380x; 1.0730x over previous
import jax
import jax.numpy as jnp
from jax import lax
from jax.experimental import pallas as pl
from jax.experimental.pallas import tpu as pltpu
from jax.experimental.pallas import tpu_sc as plsc

B = 4
S = 8192
D = 768
LANES = 16

NC = 2
NS = 16
NW = NC * NS

SPW = S // NW
CP = 8
RPU = B * CP
NU = SPW // CP
UPT = 4
NT = NU // UPT
GROUPS = D // LANES


def _body(ids_hbm, word_hbm, pos_hbm, out_hbm,
          idx_v, rows0, rows1, pos0, pos1,
          gsem0, gsem1, psem0, psem1, wsem0, wsem1):
    wid = lax.axis_index("s") * NC + lax.axis_index("c")
    soff = wid * SPW

    rows = (rows0, rows1)
    pos = (pos0, pos1)
    gsem = (gsem0, gsem1)
    psem = (psem0, psem1)
    wsem = (wsem0, wsem1)

    pltpu.sync_copy(ids_hbm.at[wid], idx_v)

    def issue_pos(u, q):
        pltpu.async_copy(
            pos_hbm.at[pl.ds(soff + u * CP, CP)], pos[q], psem[q])

    def wait_pos(q):
        pltpu.make_async_copy(pos_hbm.at[pl.ds(0, CP)], pos[q], psem[q]).wait()

    def issue_gather(u, p):
        pltpu.async_copy(word_hbm.at[idx_v.at[u]], rows[p], gsem[p])

    def wait_gather(p):
        pltpu.make_async_copy(
            word_hbm.at[idx_v.at[0]], rows[p], gsem[p]).wait()

    def issue_write(u, b, p):
        pltpu.async_copy(
            rows[p].at[pl.ds(b * CP, CP)],
            out_hbm.at[pl.ds(b * S + soff + u * CP, CP)], wsem[p])

    def wait_write(p):
        for _ in range(B):
            pltpu.make_async_copy(
                rows[p].at[pl.ds(0, CP)], out_hbm.at[pl.ds(0, CP)],
                wsem[p]).wait()

    def add_pos(u, p):
        rbuf = rows[p]
        pbuf = pos[p]

        def row_body(r, carry):
            for j in range(GROUPS):
                sl = pl.ds(j * LANES, LANES)
                pv = pbuf[r, sl]
                for b in range(B):
                    rbuf[b * CP + r, sl] = rbuf[b * CP + r, sl] + pv
            return carry

        lax.fori_loop(0, CP, row_body, 0, unroll=False)
        for b in range(B):
            issue_write(u, b, p)

    issue_pos(0, 0)
    issue_gather(0, 0)

    def iter_body(t, carry):
        for k in range(UPT):
            p = k % 2
            u = UPT * t + k
            if k == 0:
                @pl.when(t > 0)
                def _():
                    wait_write(1 - p)
            else:
                wait_write(1 - p)
            if k == UPT - 1:
                @pl.when(t < NT - 1)
                def _():
                    issue_gather(u + 1, 1 - p)
                    issue_pos(u + 1, 1 - p)
            else:
                issue_gather(u + 1, 1 - p)
                issue_pos(u + 1, 1 - p)
            wait_gather(p)
            wait_pos(p)
            add_pos(u, p)
        return carry

    lax.fori_loop(0, NT, iter_body, 0, unroll=False)

    wait_write(1)


@jax.jit
def kernel(input_ids, word_embeddings_weight, position_embeddings_weight):
    ids = jnp.reshape(input_ids.astype(jnp.int32), (B, NW, NU, CP))
    ids = jnp.reshape(jnp.transpose(ids, (1, 2, 0, 3)), (NW, NU, B * CP))
    fn = pl.kernel(
        _body,
        out_type=jax.ShapeDtypeStruct((B * S, D), jnp.float32),
        mesh=plsc.VectorSubcoreMesh(core_axis_name="c", subcore_axis_name="s"),
        scratch_types=[
            pltpu.VMEM((NU, B * CP), jnp.int32),
            pltpu.VMEM((RPU, D), jnp.float32),
            pltpu.VMEM((RPU, D), jnp.float32),
            pltpu.VMEM((CP, D), jnp.float32),
            pltpu.VMEM((CP, D), jnp.float32),
            pltpu.SemaphoreType.DMA,
            pltpu.SemaphoreType.DMA,
            pltpu.SemaphoreType.DMA,
            pltpu.SemaphoreType.DMA,
            pltpu.SemaphoreType.DMA,
            pltpu.SemaphoreType.DMA,
        ],
    )
    out = fn(ids, word_embeddings_weight, position_embeddings_weight)
    return jnp.reshape(out, (B, S, D))


# speedup vs baseline: 1.1648x; 1.0236x over previous
import jax
import jax.numpy as jnp
from jax import lax
from jax.experimental import pallas as pl
from jax.experimental.pallas import tpu as pltpu
from jax.experimental.pallas import tpu_sc as plsc

B = 4
S = 8192
D = 768
LANES = 16

NC = 2
NS = 16
NW = NC * NS

SPW = S // NW
CP = 8
RPU = B * CP
NU = SPW // CP
UPT = 4
NT = NU // UPT
GROUPS = D // LANES
NB = 4


def _body(ids_hbm, word_hbm, pos_hbm, out_hbm,
          idx_v, rows0, rows1, rows2, rows3, pos0, pos1,
          gsem0, gsem1, gsem2, gsem3, psem0, psem1,
          wsem0, wsem1, wsem2, wsem3):
    wid = lax.axis_index("s") * NC + lax.axis_index("c")
    soff = wid * SPW

    rows = (rows0, rows1, rows2, rows3)
    pos = (pos0, pos1)
    gsem = (gsem0, gsem1, gsem2, gsem3)
    psem = (psem0, psem1)
    wsem = (wsem0, wsem1, wsem2, wsem3)

    pltpu.sync_copy(ids_hbm.at[wid], idx_v)

    def issue_pos(u, q):
        pltpu.async_copy(
            pos_hbm.at[pl.ds(soff + u * CP, CP)], pos[q], psem[q])

    def wait_pos(q):
        pltpu.make_async_copy(pos_hbm.at[pl.ds(0, CP)], pos[q], psem[q]).wait()

    def issue_gather(u, p):
        pltpu.async_copy(word_hbm.at[idx_v.at[u]], rows[p], gsem[p])

    def wait_gather(p):
        pltpu.make_async_copy(
            word_hbm.at[idx_v.at[0]], rows[p], gsem[p]).wait()

    def issue_write(u, b, p):
        pltpu.async_copy(
            rows[p].at[pl.ds(b * CP, CP)],
            out_hbm.at[pl.ds(b * S + soff + u * CP, CP)], wsem[p])

    def wait_write(p):
        for _ in range(B):
            pltpu.make_async_copy(
                rows[p].at[pl.ds(0, CP)], out_hbm.at[pl.ds(0, CP)],
                wsem[p]).wait()

    def add_pos(u, p, q):
        rbuf = rows[p]
        pbuf = pos[q]

        def row_body(r, carry):
            for j in range(GROUPS):
                sl = pl.ds(j * LANES, LANES)
                pv = pbuf[r, sl]
                for b in range(B):
                    rbuf[b * CP + r, sl] = rbuf[b * CP + r, sl] + pv
            return carry

        lax.fori_loop(0, CP, row_body, 0, unroll=False)
        for b in range(B):
            issue_write(u, b, p)

    issue_pos(0, 0)
    issue_gather(0, 0)
    issue_gather(1, 1)

    def iter_body(t, carry):
        for k in range(UPT):
            p = k
            q = k % 2
            u = UPT * t + k
            if k == UPT - 1:
                @pl.when(t < NT - 1)
                def _():
                    issue_pos(u + 1, 1 - q)
            else:
                issue_pos(u + 1, 1 - q)
            if k < 2:
                @pl.when(t > 0)
                def _():
                    wait_write((k + 2) % NB)
                issue_gather(u + 2, (k + 2) % NB)
            else:
                wait_write((k + 2) % NB)
                @pl.when(t < NT - 1)
                def _():
                    issue_gather(u + 2, (k + 2) % NB)
            wait_gather(p)
            wait_pos(q)
            add_pos(u, p, q)
        return carry

    lax.fori_loop(0, NT, iter_body, 0, unroll=False)

    wait_write(2)
    wait_write(3)


@jax.jit
def kernel(input_ids, word_embeddings_weight, position_embeddings_weight):
    ids = jnp.reshape(input_ids.astype(jnp.int32), (B, NW, NU, CP))
    ids = jnp.reshape(jnp.transpose(ids, (1, 2, 0, 3)), (NW, NU, B * CP))
    fn = pl.kernel(
        _body,
        out_type=jax.ShapeDtypeStruct((B * S, D), jnp.float32),
        mesh=plsc.VectorSubcoreMesh(core_axis_name="c", subcore_axis_name="s"),
        scratch_types=[
            pltpu.VMEM((NU, B * CP), jnp.int32),
            pltpu.VMEM((RPU, D), jnp.float32),
            pltpu.VMEM((RPU, D), jnp.float32),
            pltpu.VMEM((RPU, D), jnp.float32),
            pltpu.VMEM((RPU, D), jnp.float32),
            pltpu.VMEM((CP, D), jnp.float32),
            pltpu.VMEM((CP, D), jnp.float32),
            pltpu.SemaphoreType.DMA,
            pltpu.SemaphoreType.DMA,
            pltpu.SemaphoreType.DMA,
            pltpu.SemaphoreType.DMA,
            pltpu.SemaphoreType.DMA,
            pltpu.SemaphoreType.DMA,
            pltpu.SemaphoreType.DMA,
            pltpu.SemaphoreType.DMA,
            pltpu.SemaphoreType.DMA,
            pltpu.SemaphoreType.DMA,
        ],
    )
    out = fn(ids, word_embeddings_weight, position_embeddings_weight)
    return jnp.reshape(out, (B, S, D))
